# baseline (speedup 1.0000x reference)
import jax
import jax.numpy as jnp
from jax.experimental import pallas as pl
from jax.experimental.pallas import tpu as pltpu

T = 8
DIM = 1024
INTER = 2816
E = 8
TOPK = 2

BI = 1408
NJ = INTER // BI
HD = DIM // 2


def _ffn_kernel(order_ref, ew_ref, idx_ref, x_ref,
                w1a_ref, w1b_ref, w3a_ref, w3b_ref, w2a_ref, w2b_ref,
                out_ref):
    i = pl.program_id(0)
    j = pl.program_id(1)

    @pl.when(jnp.logical_and(i == 0, j == 0))
    def _init():
        out_ref[...] = jnp.zeros_like(out_ref)

    e = order_ref[i]

    ew = ew_ref[...]
    ewn = ew / jnp.sum(ew, axis=-1, keepdims=True)
    idx = idx_ref[...]
    scale = jnp.sum(jnp.where(idx == e, ewn, 0.0), axis=-1, keepdims=True)

    x = x_ref[...]
    xa = x[:, :HD]
    xb = x[:, HD:]

    def mm_t(a, b):
        return jax.lax.dot_general(a, b, (((1,), (1,)), ((), ())),
                                   preferred_element_type=jnp.float32)

    h1 = mm_t(xa, w1a_ref[0]) + mm_t(xb, w1b_ref[0])
    h3 = mm_t(xa, w3a_ref[0]) + mm_t(xb, w3b_ref[0])
    h = (h1 * jax.nn.sigmoid(h1)) * h3 * scale

    out_ref[:, :HD] += mm_t(h, w2a_ref[0])
    out_ref[:, HD:] += mm_t(h, w2b_ref[0])


def kernel(x, expert_weights, expert_indices, w1, w2, w3):
    idx = expert_indices.astype(jnp.int32)

    used = jnp.zeros((E,), jnp.int32).at[idx.reshape(-1)].set(1, mode="drop")
    order = jnp.argsort(-used, stable=True).astype(jnp.int32)
    num_used = jnp.sum(used)

    def half_k(h):
        return pl.BlockSpec((1, BI, HD), lambda i, j, order, h=h: (order[i], j, h))

    def half_r(h):
        return pl.BlockSpec((1, HD, BI), lambda i, j, order, h=h: (order[i], h, j))

    grid_spec = pltpu.PrefetchScalarGridSpec(
        num_scalar_prefetch=1,
        grid=(num_used, NJ),
        in_specs=[
            pl.BlockSpec((T, TOPK), lambda i, j, order: (0, 0)),
            pl.BlockSpec((T, TOPK), lambda i, j, order: (0, 0)),
            pl.BlockSpec((T, DIM), lambda i, j, order: (0, 0)),
            half_k(0), half_k(1),
            half_k(0), half_k(1),
            half_r(0), half_r(1),
        ],
        out_specs=pl.BlockSpec((T, DIM), lambda i, j, order: (0, 0)),
    )
    return pl.pallas_call(
        _ffn_kernel,
        grid_spec=grid_spec,
        out_shape=jax.ShapeDtypeStruct((T, DIM), jnp.float32),
    )(order, expert_weights, idx, x, w1, w1, w3, w3, w2, w2)


# speedup vs baseline: 1.0067x; 1.0067x over previous
import jax
import jax.numpy as jnp
from jax.experimental import pallas as pl
from jax.experimental.pallas import tpu as pltpu

T = 8
DIM = 1024
INTER = 2816
E = 8
TOPK = 2

BI = 1408
NJ = INTER // BI


def _ffn_kernel(order_ref, ew_ref, idx_ref, x_ref,
                w1_ref, w3_ref, w2_ref, out_ref):
    i = pl.program_id(0)
    j = pl.program_id(1)

    @pl.when(jnp.logical_and(i == 0, j == 0))
    def _init():
        out_ref[...] = jnp.zeros_like(out_ref)

    e = order_ref[i]

    ew = ew_ref[...]
    ewn = ew / jnp.sum(ew, axis=-1, keepdims=True)
    idx = idx_ref[...]
    scale = jnp.sum(jnp.where(idx == e, ewn, 0.0), axis=-1, keepdims=True)

    x = x_ref[...]
    w1 = w1_ref[0]
    w3 = w3_ref[0]
    w2 = w2_ref[0]

    h1 = jax.lax.dot_general(x, w1, (((1,), (1,)), ((), ())),
                             preferred_element_type=jnp.float32)
    h3 = jax.lax.dot_general(x, w3, (((1,), (1,)), ((), ())),
                             preferred_element_type=jnp.float32)
    h = (h1 * jax.nn.sigmoid(h1)) * h3 * scale

    contrib = jax.lax.dot_general(h, w2, (((1,), (1,)), ((), ())),
                                  preferred_element_type=jnp.float32)
    out_ref[...] += contrib


def kernel(x, expert_weights, expert_indices, w1, w2, w3):
    idx = expert_indices.astype(jnp.int32)

    used = jnp.zeros((E,), jnp.int32).at[idx.reshape(-1)].set(1, mode="drop")
    order = jnp.argsort(-used, stable=True).astype(jnp.int32)
    num_used = jnp.sum(used)

    grid_spec = pltpu.PrefetchScalarGridSpec(
        num_scalar_prefetch=1,
        grid=(num_used, NJ),
        in_specs=[
            pl.BlockSpec((T, TOPK), lambda i, j, order: (0, 0)),
            pl.BlockSpec((T, TOPK), lambda i, j, order: (0, 0)),
            pl.BlockSpec((T, DIM), lambda i, j, order: (0, 0)),
            pl.BlockSpec((1, BI, DIM), lambda i, j, order: (order[i], j, 0)),
            pl.BlockSpec((1, BI, DIM), lambda i, j, order: (order[i], j, 0)),
            pl.BlockSpec((1, DIM, BI), lambda i, j, order: (order[i], 0, j)),
        ],
        out_specs=pl.BlockSpec((T, DIM), lambda i, j, order: (0, 0)),
    )
    return pl.pallas_call(
        _ffn_kernel,
        grid_spec=grid_spec,
        out_shape=jax.ShapeDtypeStruct((T, DIM), jnp.float32),
    )(order, expert_weights, idx, x, w1, w3, w2)
